# baseline (speedup 1.0000x reference)
import functools

import jax
import jax.numpy as jnp
from jax import lax
from jax.experimental import pallas as pl
from jax.experimental.pallas import tpu as pltpu
from jax.experimental.pallas import tpu_sc as plsc

VOCAB = 1000000
DIM = 64
SEQ = 200
BATCH = 4096

NC = 2
NS = 16
NW = NC * NS
G = BATCH // NW

_mesh = plsc.VectorSubcoreMesh(core_axis_name="c", subcore_axis_name="s")


def _iotas():
    base = lax.iota(jnp.int32, 16)
    return [base + (16 * j) for j in range(4)]


@functools.partial(
    pl.kernel,
    mesh=_mesh,
    out_type=jax.ShapeDtypeStruct((SEQ, 8, 32, 8, 128), jnp.float32),
    scratch_types=[
        pltpu.VMEM((SEQ, G), jnp.int32),
        pltpu.VMEM((G, DIM), jnp.float32),
        pltpu.VMEM((G, DIM), jnp.float32),
        pltpu.VMEM((8, 8, 128), jnp.float32),
        pltpu.VMEM((8, 8, 128), jnp.float32),
        pltpu.SemaphoreType.DMA,
        pltpu.SemaphoreType.DMA,
        pltpu.SemaphoreType.DMA,
    ],
    compiler_params=pltpu.CompilerParams(
        use_tc_tiling_on_sc=False, needs_layout_passes=False
    ),
)
def _sc_gather(table_hbm, idx_hbm, out_hbm, idx_v, gb0, gb1, sb0, sb1,
               gsem, wsem0, wsem1):
    wid = lax.axis_index("s") * NC + lax.axis_index("c")
    wb = wid * G
    pltpu.sync_copy(idx_hbm.at[:, pl.ds(wb, G)], idx_v)
    iot = _iotas()
    gbufs = (gb0, gb1)
    sbufs = (sb0, sb1)
    wsems = (wsem0, wsem1)

    def fire_gather(s, p):
        @pl.when(s < SEQ)
        def _():
            pltpu.async_copy(table_hbm.at[idx_v.at[s]], gbufs[p], gsem)

    def wait_gather(p):
        pltpu.make_async_copy(
            table_hbm.at[pl.ds(0, G)], gbufs[p], gsem
        ).wait()

    def transpose(p):
        gb = gbufs[p]
        sb = sbufs[p]

        def body(dt, carry):
            for dl in range(8):
                d = jnp.full((16,), dt * 8 + dl, jnp.int32)
                for j in range(8):
                    sb[dt, dl, pl.ds(16 * j, 16)] = plsc.load_gather(
                        gb, [iot[j % 4] + (64 if j >= 4 else 0), d]
                    )
            return carry

        lax.fori_loop(0, 8, body, 0)

    def fire_write(s, p):
        pltpu.async_copy(sbufs[p], out_hbm.at[s, :, wid, :, :], wsems[p])

    def wait_write(p):
        pltpu.make_async_copy(
            sbufs[p], out_hbm.at[0, :, wid, :, :], wsems[p]
        ).wait()

    fire_gather(0, 0)
    fire_gather(1, 1)
    wait_gather(0)
    transpose(0)
    fire_gather(2, 0)
    fire_write(0, 0)
    wait_gather(1)
    transpose(1)
    fire_gather(3, 1)
    fire_write(1, 1)

    def body(c, carry):
        for q in range(2):
            s = 2 * c + q
            p = q
            wait_gather(p)
            wait_write(p)
            transpose(p)
            fire_gather(s + 2, p)
            fire_write(s, p)
        return carry

    lax.fori_loop(1, SEQ // 2, body, 0)
    wait_write(0)
    wait_write(1)


def kernel(src_input, word_lut):
    idx = src_input.reshape(SEQ, BATCH)
    out5 = _sc_gather(word_lut, idx)
    return out5.transpose(0, 2, 4, 1, 3).reshape(SEQ, BATCH, DIM)


# speedup vs baseline: 1.5267x; 1.5267x over previous
import functools

import jax
import jax.numpy as jnp
from jax import lax
from jax.experimental import pallas as pl
from jax.experimental.pallas import tpu as pltpu
from jax.experimental.pallas import tpu_sc as plsc

VOCAB = 1000000
DIM = 64
SEQ = 200
BATCH = 4096

NC = 2
NS = 16
NW = NC * NS
G = BATCH // NW
K = 4
STEPS = SEQ // (2 * K)

_mesh = plsc.VectorSubcoreMesh(core_axis_name="c", subcore_axis_name="s")


@functools.partial(
    pl.kernel,
    mesh=_mesh,
    out_type=jax.ShapeDtypeStruct((SEQ, BATCH, DIM), jnp.float32),
    scratch_types=[
        pltpu.VMEM((SEQ, G), jnp.int32),
        pltpu.VMEM((K, G, DIM), jnp.float32),
        pltpu.VMEM((K, G, DIM), jnp.float32),
        pltpu.SemaphoreType.DMA,
        pltpu.SemaphoreType.DMA,
        pltpu.SemaphoreType.DMA,
    ],
    compiler_params=pltpu.CompilerParams(use_tc_tiling_on_sc=False),
)
def _sc_gather(table_hbm, idx_hbm, out_hbm, idx_v, rows0, rows1, gsem, wsem0, wsem1):
    wid = lax.axis_index("s") * NC + lax.axis_index("c")
    wb = wid * G
    pltpu.sync_copy(idx_hbm.at[:, pl.ds(wb, G)], idx_v)

    def fire_and_drain(t, rows_v):
        handles = [
            pltpu.async_copy(
                table_hbm.at[idx_v.at[t * K + k]],
                rows_v.at[k],
                gsem,
            )
            for k in range(K)
        ]
        for h in handles:
            h.wait()

    def start_write(t, rows_v, wsem):
        pltpu.async_copy(
            rows_v, out_hbm.at[pl.ds(t * K, K), pl.ds(wb, G), :], wsem
        )

    def wait_write(rows_v, wsem):
        pltpu.make_async_copy(
            rows_v, out_hbm.at[pl.ds(0, K), pl.ds(wb, G), :], wsem
        ).wait()

    fire_and_drain(0, rows0)
    start_write(0, rows0, wsem0)
    fire_and_drain(1, rows1)
    start_write(1, rows1, wsem1)

    def body(c, carry):
        t0 = 2 * c
        wait_write(rows0, wsem0)
        fire_and_drain(t0, rows0)
        start_write(t0, rows0, wsem0)
        wait_write(rows1, wsem1)
        fire_and_drain(t0 + 1, rows1)
        start_write(t0 + 1, rows1, wsem1)
        return carry

    lax.fori_loop(1, STEPS, body, 0)

    wait_write(rows0, wsem0)
    wait_write(rows1, wsem1)


def kernel(src_input, word_lut):
    idx = src_input.reshape(SEQ, BATCH)
    return _sc_gather(word_lut, idx)


# speedup vs baseline: 1.5300x; 1.0021x over previous
import functools

import jax
import jax.numpy as jnp
from jax import lax
from jax.experimental import pallas as pl
from jax.experimental.pallas import tpu as pltpu
from jax.experimental.pallas import tpu_sc as plsc

VOCAB = 1000000
DIM = 64
SEQ = 200
BATCH = 4096

NC = 2
NS = 16
NW = NC * NS
G = BATCH // NW
K = 4
TOTAL = SEQ // K

_mesh = plsc.VectorSubcoreMesh(core_axis_name="c", subcore_axis_name="s")


@functools.partial(
    pl.kernel,
    mesh=_mesh,
    out_type=jax.ShapeDtypeStruct((SEQ, BATCH, DIM), jnp.float32),
    scratch_types=[
        pltpu.VMEM((SEQ, G), jnp.int32),
        pltpu.VMEM((K, G, DIM), jnp.float32),
        pltpu.VMEM((K, G, DIM), jnp.float32),
        pltpu.SemaphoreType.DMA,
        pltpu.SemaphoreType.DMA,
        pltpu.SemaphoreType.DMA,
        pltpu.SemaphoreType.DMA,
    ],
    compiler_params=pltpu.CompilerParams(use_tc_tiling_on_sc=False),
)
def _sc_gather(table_hbm, idx_hbm, out_hbm, idx_v, rows0, rows1,
               gsem0, gsem1, wsem0, wsem1):
    wid = lax.axis_index("s") * NC + lax.axis_index("c")
    wb = wid * G
    pltpu.sync_copy(idx_hbm.at[:, pl.ds(wb, G)], idx_v)

    rows = (rows0, rows1)
    gsems = (gsem0, gsem1)
    wsems = (wsem0, wsem1)

    def fire(t, p):
        for k in range(K):
            pltpu.async_copy(
                table_hbm.at[idx_v.at[t * K + k]],
                rows[p].at[k],
                gsems[p],
            )

    def drain(p):
        for k in range(K):
            pltpu.make_async_copy(
                table_hbm.at[pl.ds(0, G)], rows[p].at[k], gsems[p]
            ).wait()

    def start_write(t, p):
        pltpu.async_copy(
            rows[p], out_hbm.at[pl.ds(t * K, K), pl.ds(wb, G), :], wsems[p]
        )

    def wait_write(p):
        pltpu.make_async_copy(
            rows[p], out_hbm.at[pl.ds(0, K), pl.ds(wb, G), :], wsems[p]
        ).wait()

    def step(t, p, fire_ahead):
        drain(p)
        start_write(t, p)
        if fire_ahead:
            wait_write(p)
            fire(t + 2, p)

    fire(0, 0)
    fire(1, 1)
    step(0, 0, True)
    step(1, 1, True)

    def body(c, carry):
        t0 = 2 * c
        step(t0, 0, True)
        step(t0 + 1, 1, True)
        return carry

    lax.fori_loop(1, TOTAL // 2 - 1, body, 0)

    step(TOTAL - 2, 0, False)
    step(TOTAL - 1, 1, False)
    wait_write(0)
    wait_write(1)


def kernel(src_input, word_lut):
    idx = src_input.reshape(SEQ, BATCH)
    return _sc_gather(word_lut, idx)


# speedup vs baseline: 1.5317x; 1.0011x over previous
import functools

import jax
import jax.numpy as jnp
from jax import lax
from jax.experimental import pallas as pl
from jax.experimental.pallas import tpu as pltpu
from jax.experimental.pallas import tpu_sc as plsc

VOCAB = 1000000
DIM = 64
SEQ = 200
BATCH = 4096

NC = 2
NS = 16
NW = NC * NS
G = BATCH // NW
K = 4
TOTAL = SEQ // K

_mesh = plsc.VectorSubcoreMesh(core_axis_name="c", subcore_axis_name="s")


@functools.partial(
    pl.kernel,
    mesh=_mesh,
    out_type=jax.ShapeDtypeStruct((SEQ, BATCH, DIM), jnp.float32),
    scratch_types=[
        pltpu.VMEM((SEQ, G), jnp.int32),
        pltpu.VMEM((K, G, DIM), jnp.float32),
        pltpu.VMEM((K, G, DIM), jnp.float32),
        pltpu.SemaphoreType.DMA,
        pltpu.SemaphoreType.DMA,
        pltpu.SemaphoreType.DMA,
        pltpu.SemaphoreType.DMA,
    ],
    compiler_params=pltpu.CompilerParams(
        use_tc_tiling_on_sc=False, needs_layout_passes=False
    ),
)
def _sc_gather(table_hbm, idx_hbm, out_hbm, idx_v, rows0, rows1,
               gsem0, gsem1, wsem0, wsem1):
    wid = lax.axis_index("s") * NC + lax.axis_index("c")
    wb = wid * G
    pltpu.sync_copy(idx_hbm.at[:, pl.ds(wb, G)], idx_v)

    rows = (rows0, rows1)
    gsems = (gsem0, gsem1)
    wsems = (wsem0, wsem1)

    def fire(t, p):
        for k in range(K):
            pltpu.async_copy(
                table_hbm.at[idx_v.at[t * K + k]],
                rows[p].at[k],
                gsems[p],
            )

    def drain(p):
        for k in range(K):
            pltpu.make_async_copy(
                table_hbm.at[pl.ds(0, G)], rows[p].at[k], gsems[p]
            ).wait()

    def start_write(t, p):
        pltpu.async_copy(
            rows[p], out_hbm.at[pl.ds(t * K, K), pl.ds(wb, G), :], wsems[p]
        )

    def wait_write(p):
        pltpu.make_async_copy(
            rows[p], out_hbm.at[pl.ds(0, K), pl.ds(wb, G), :], wsems[p]
        ).wait()

    def step(t, p, fire_ahead):
        drain(p)
        start_write(t, p)
        if fire_ahead:
            wait_write(p)
            fire(t + 2, p)

    fire(0, 0)
    fire(1, 1)
    step(0, 0, True)
    step(1, 1, True)

    def body(c, carry):
        t0 = 2 * c
        step(t0, 0, True)
        step(t0 + 1, 1, True)
        return carry

    lax.fori_loop(1, TOTAL // 2 - 1, body, 0)

    step(TOTAL - 2, 0, False)
    step(TOTAL - 1, 1, False)
    wait_write(0)
    wait_write(1)


def kernel(src_input, word_lut):
    idx = src_input.reshape(SEQ, BATCH)
    return _sc_gather(word_lut, idx)
